# baseline (device time: 54045 ns/iter reference)
import functools

import jax
import jax.numpy as jnp
from jax import lax
from jax.experimental import pallas as pl
from jax.experimental.pallas import tpu as pltpu

Z = 4
K = 8


def kernel(x):
    m, n = x.shape
    rows = m // K

    def body(x_ref, out_ref, edge_rx, ctr_rx, mid_ref,
             e_send, e_rx, c_send, c_rx, r_send, r_rx):
        my_x = lax.axis_index("x")
        my_y = lax.axis_index("y")
        my_z = lax.axis_index("z")
        is_edge = (my_z == 0) | (my_z == Z - 1)

        def sig(sem, z_tgt):
            pl.semaphore_signal(
                sem, inc=1,
                device_id=(my_x, my_y, z_tgt),
                device_id_type=pl.DeviceIdType.MESH,
            )

        def line_barrier(sem):
            @pl.when(my_z > 0)
            def _():
                sig(sem, my_z - 1)

            @pl.when(my_z < Z - 1)
            def _():
                sig(sem, my_z + 1)

            @pl.when(is_edge)
            def _():
                pl.semaphore_wait(sem, 1)

            @pl.when(jnp.logical_not(is_edge))
            def _():
                pl.semaphore_wait(sem, 2)

        line_barrier(pltpu.get_barrier_semaphore())

        def rdma(src, dst, ssem, rsem, z_tgt):
            return pltpu.make_async_remote_copy(
                src_ref=src, dst_ref=dst, send_sem=ssem, recv_sem=rsem,
                device_id=(my_x, my_y, z_tgt),
                device_id_type=pl.DeviceIdType.MESH,
            )

        @pl.when(is_edge)
        def _():
            ctr = jnp.where(my_z == 0, 1, Z - 2)
            sends = []
            for k in range(K):
                sl = pl.ds(k * rows, rows)
                r = rdma(x_ref.at[sl, :], edge_rx.at[k],
                         e_send.at[k], e_rx.at[k], ctr)
                r.start()
                sends.append(r)
            for k in range(K):
                sl = pl.ds(k * rows, rows)
                rr = rdma(x_ref.at[sl, :], out_ref.at[sl, :],
                          r_send.at[k], r_rx.at[k], ctr)
                rr.wait_recv()
            for r in sends:
                r.wait_send()

        @pl.when(jnp.logical_not(is_edge))
        def _():
            edge = jnp.where(my_z == 1, 0, Z - 1)
            peer = jnp.where(my_z == 1, 2, 1)
            csends = []
            rsends = []
            for k in range(K):
                sl = pl.ds(k * rows, rows)
                er = rdma(x_ref.at[sl, :], edge_rx.at[k],
                          e_send.at[k], e_rx.at[k], edge)
                er.wait_recv()
                mid_ref[sl, :] = edge_rx[k] + x_ref[sl, :]
                cs = rdma(mid_ref.at[sl, :], ctr_rx.at[k],
                          c_send.at[k], c_rx.at[k], peer)
                cs.start()
                csends.append(cs)
            for k in range(K):
                sl = pl.ds(k * rows, rows)
                csends[k].wait_recv()
                out_ref[sl, :] = mid_ref[sl, :] + ctr_rx[k]
                rs = rdma(out_ref.at[sl, :], out_ref.at[sl, :],
                          r_send.at[k], r_rx.at[k], edge)
                rs.start()
                rsends.append(rs)
            for r in csends:
                r.wait_send()
            for r in rsends:
                r.wait_send()

        @functools.partial(
            pl.run_scoped, second_barrier=pltpu.SemaphoreType.REGULAR
        )
        def _(second_barrier):
            line_barrier(second_barrier)

    return pl.pallas_call(
        body,
        out_shape=jax.ShapeDtypeStruct((m, n), x.dtype),
        in_specs=[pl.BlockSpec(memory_space=pltpu.VMEM)],
        out_specs=pl.BlockSpec(memory_space=pltpu.VMEM),
        scratch_shapes=[
            pltpu.VMEM((K, rows, n), x.dtype),
            pltpu.VMEM((K, rows, n), x.dtype),
            pltpu.VMEM((m, n), x.dtype),
            pltpu.SemaphoreType.DMA((K,)),
            pltpu.SemaphoreType.DMA((K,)),
            pltpu.SemaphoreType.DMA((K,)),
            pltpu.SemaphoreType.DMA((K,)),
            pltpu.SemaphoreType.DMA((K,)),
            pltpu.SemaphoreType.DMA((K,)),
        ],
        compiler_params=pltpu.CompilerParams(collective_id=0),
    )(x)
